# Initial kernel scaffold; baseline (speedup 1.0000x reference)
import jax
import jax.numpy as jnp
from jax.experimental import pallas as pl


def kernel(pts, w1, g1, b1, w2, g2, b2, w3, g3, b3, w4, g4, b4, w5, g5, b5):
    raise NotImplementedError("write your pallas kernel here")


# speedup vs baseline: 8.0101x; 8.0101x over previous
import functools

import numpy as np

import jax
import jax.numpy as jnp
from jax import lax
from jax.experimental import pallas as pl
from jax.experimental.pallas import tpu as pltpu
from jax.experimental.pallas import tpu_sc as plsc

B = 8
N = 1024
K = 20
R = 256
BN = B * N
CP = 128
SQ = np.float32(np.sqrt(np.float32(1.0 + 1e-5)))
NEG = -3e38

SC_NC = 2
SC_NS = 16
SC_NW = SC_NC * SC_NS
PW = BN // SC_NW
CHP = 128
NCH = PW // CHP


def _bn_lrelu(y, g, b):
    y = g * (y / SQ) + b
    return jnp.where(y >= 0, y, 0.2 * y)


def _dot_t(a, b):
    return lax.dot_general(a, b, (((1,), (1,)), ((), ())),
                           preferred_element_type=jnp.float32)


def _topk_body(x_ref, idx_ref):
    b = pl.program_id(0)
    r = pl.program_id(1)
    xtf = x_ref[0]
    rows = x_ref[0, pl.ds(r * R, R), :]
    d = _dot_t(rows, xtf)
    xxr = jnp.sum(rows * rows, axis=1, keepdims=True)
    xxa = jnp.sum(xtf * xtf, axis=1)[None, :]
    vals = 2.0 * d - xxr - xxa
    iota = lax.broadcasted_iota(jnp.int32, (R, N), 1)
    for j in range(K):
        m = jnp.max(vals, axis=1, keepdims=True)
        am = jnp.min(jnp.where(vals == m, iota, N), axis=1, keepdims=True)
        vals = jnp.where(iota == am, NEG, vals)
        idx_ref[0, j, :] = am[:, 0] + b * N


def _topk(x):
    return pl.pallas_call(
        _topk_body,
        grid=(B, N // R),
        in_specs=[pl.BlockSpec((1, N, CP), lambda b, r: (b, 0, 0))],
        out_specs=pl.BlockSpec((1, K, R), lambda b, r: (b, 0, r)),
        out_shape=jax.ShapeDtypeStruct((B, K, N), jnp.int32),
    )(x)


NBUF = 4


def _gather_body(x_hbm, idx_hbm, out_hbm, idx_v, rows_v, gsems, wsems):
    wid = lax.axis_index("s") * SC_NC + lax.axis_index("c")
    base = wid * PW
    for j in range(K):
        pltpu.sync_copy(idx_hbm.at[j, pl.ds(base, PW)], idx_v.at[j])

    tasks = [(j, c) for j in range(K) for c in range(NCH)]

    def gcopy(t, bi):
        j, c = tasks[t]
        return pltpu.make_async_copy(
            x_hbm.at[idx_v.at[j, pl.ds(c * CHP, CHP)]], rows_v.at[bi],
            gsems[bi])

    def wcopy(t, bi):
        j, c = tasks[t]
        return pltpu.make_async_copy(
            rows_v.at[bi], out_hbm.at[j, pl.ds(base + c * CHP, CHP)],
            wsems[bi])

    nt = len(tasks)
    for t in range(min(NBUF - 1, nt)):
        gcopy(t, t % NBUF).start()
    for t in range(nt):
        bi = t % NBUF
        gcopy(t, bi).wait()
        wcopy(t, bi).start()
        ng = t + NBUF - 1
        if ng < nt:
            nbi = ng % NBUF
            if ng >= NBUF:
                wcopy(ng - NBUF, nbi).wait()
            gcopy(ng, nbi).start()
    for t in range(max(0, nt - NBUF), nt):
        wcopy(t, t % NBUF).wait()


def _gather(x_flat, idx):
    mesh = plsc.VectorSubcoreMesh(core_axis_name="c", subcore_axis_name="s")
    return pl.kernel(
        _gather_body,
        out_type=jax.ShapeDtypeStruct((K, BN, CP), jnp.float32),
        mesh=mesh,
        scratch_types=[
            pltpu.VMEM((K, PW), jnp.int32),
            pltpu.VMEM((NBUF, CHP, CP), jnp.float32),
            [pltpu.SemaphoreType.DMA] * NBUF,
            [pltpu.SemaphoreType.DMA] * NBUF,
        ],
    )(x_flat, idx)


def _edge_body(feats_ref, x_ref, w_ref, g_ref, b_ref, out_ref):
    r = pl.program_id(1)
    center = x_ref[0, pl.ds(r * R, R), :]
    g = g_ref[...]
    bb = b_ref[...]
    acc = None
    for j in range(K):
        fj = feats_ref[j, 0]
        fcat = jnp.concatenate([fj - center, center], axis=1)
        y = _bn_lrelu(_dot_t(fcat, w_ref[...]), g, bb)
        acc = y if acc is None else jnp.maximum(acc, y)
    out_ref[0] = acc


def _edge(feats, x, w, g, b):
    coutp = w.shape[0]
    return pl.pallas_call(
        _edge_body,
        grid=(B, N // R),
        in_specs=[
            pl.BlockSpec((K, 1, R, CP), lambda b, r: (0, b, r, 0)),
            pl.BlockSpec((1, N, CP), lambda b, r: (b, 0, 0)),
            pl.BlockSpec((coutp, 2 * CP), lambda b, r: (0, 0)),
            pl.BlockSpec((1, coutp), lambda b, r: (0, 0)),
            pl.BlockSpec((1, coutp), lambda b, r: (0, 0)),
        ],
        out_specs=pl.BlockSpec((1, R, coutp), lambda b, r: (b, r, 0)),
        out_shape=jax.ShapeDtypeStruct((B, N, coutp), jnp.float32),
    )(feats.reshape(K, B, N, CP), x, w, g, b)


def _final_body(x1, x2, x3, x4, w5_ref, g5_ref, b5_ref, out_ref):
    xc = jnp.concatenate([x1[0][:, :64], x2[0][:, :64], x3[0], x4[0]], axis=1)
    y = _dot_t(xc, w5_ref[...])
    out_ref[0] = _bn_lrelu(y, g5_ref[...], b5_ref[...])


def _final(x1, x2, x3, x4, w5, g5, b5):
    rspec = lambda c: pl.BlockSpec((1, R, c), lambda b, r: (b, r, 0))
    gspec = pl.BlockSpec((1, 1024), lambda b, r: (0, 0))
    return pl.pallas_call(
        _final_body,
        grid=(B, N // R),
        in_specs=[rspec(CP), rspec(CP), rspec(CP), rspec(256),
                  pl.BlockSpec((1024, 512), lambda b, r: (0, 0)),
                  gspec, gspec],
        out_specs=pl.BlockSpec((1, R, 1024), lambda b, r: (b, r, 0)),
        out_shape=jax.ShapeDtypeStruct((B, N, 1024), jnp.float32),
    )(x1, x2, x3, x4, w5, g5, b5)


def _layer(x, w, g, b, c, coutp):
    cout = w.shape[0]
    wp = jnp.zeros((coutp, 2 * CP), jnp.float32)
    wp = wp.at[:cout, :c].set(w[:, :c])
    wp = wp.at[:cout, CP:CP + c].set(w[:, c:])
    gp = jnp.ones((1, coutp), jnp.float32).at[:, :cout].set(g[None, :])
    bp = jnp.zeros((1, coutp), jnp.float32).at[:, :cout].set(b[None, :])
    idx = _topk(x)
    feats = _gather(x.reshape(BN, CP),
                    jnp.transpose(idx, (1, 0, 2)).reshape(K, BN))
    return _edge(feats, x, wp, gp, bp)


def kernel(pts, w1, g1, b1, w2, g2, b2, w3, g3, b3, w4, g4, b4, w5, g5, b5):
    f32 = jnp.float32
    x0 = jnp.pad(pts[:, :, :3].astype(f32), ((0, 0), (0, 0), (0, CP - 3)))
    x1 = _layer(x0, w1.astype(f32), g1, b1, c=3, coutp=CP)
    x2 = _layer(x1, w2.astype(f32), g2, b2, c=64, coutp=CP)
    x3 = _layer(x2, w3.astype(f32), g3, b3, c=64, coutp=CP)
    x4 = _layer(x3, w4.astype(f32), g4, b4, c=128, coutp=256)
    y = _final(x1, x2, x3, x4, w5.astype(f32),
               g5.reshape(1, -1).astype(f32), b5.reshape(1, -1).astype(f32))
    return jnp.transpose(y, (0, 2, 1))
